# baseline (speedup 1.0000x reference)
import functools

import jax
import jax.numpy as jnp
from jax import lax
from jax.experimental import pallas as pl
from jax.experimental.pallas import tpu as pltpu
from jax.experimental.pallas import tpu_sc as plsc

_B = 4
_L = 4096
_H = 1024
_NC = 2
_NS = 16
_NW = _NC * _NS
_ROWS_PER_W = _L // _NW
_CHUNK = 32
_VECS_PER_ROW = _H // 16


def _body(x_hbm, pos_hbm, out_hbm, xbuf, posbuf, sem):
    wid = lax.axis_index("s") * _NC + lax.axis_index("c")
    row0 = wid * _ROWS_PER_W

    def add_rows(i, _):
        for u in range(_VECS_PER_ROW):
            sl = pl.ds(u * 16, 16)
            xbuf[i, sl] = xbuf[i, sl] + posbuf[i, sl]
        return 0

    for c in range(_ROWS_PER_W // _CHUNK):
        r = row0 + c * _CHUNK
        pltpu.sync_copy(pos_hbm.at[pl.ds(r, _CHUNK)], posbuf)
        for b in range(_B):
            pltpu.sync_copy(x_hbm.at[b, pl.ds(r, _CHUNK)], xbuf)
            lax.fori_loop(0, _CHUNK, add_rows, 0)
            pltpu.sync_copy(xbuf, out_hbm.at[b, pl.ds(r, _CHUNK)])


def kernel(x, pos_table):
    mesh = plsc.VectorSubcoreMesh(core_axis_name="c", subcore_axis_name="s")
    k = functools.partial(
        pl.kernel,
        mesh=mesh,
        out_type=jax.ShapeDtypeStruct((_B, _L, _H), jnp.float32),
        scratch_types=[
            pltpu.VMEM((_CHUNK, _H), jnp.float32),
            pltpu.VMEM((_CHUNK, _H), jnp.float32),
            pltpu.SemaphoreType.DMA,
        ],
    )(_body)
    return k(x, pos_table)


# speedup vs baseline: 1.2482x; 1.2482x over previous
import functools

import jax
import jax.numpy as jnp
from jax import lax
from jax.experimental import pallas as pl
from jax.experimental.pallas import tpu as pltpu
from jax.experimental.pallas import tpu_sc as plsc

_B = 4
_L = 4096
_H = 1024
_NC = 2
_NS = 16
_NW = _NC * _NS
_ROWS_PER_W = _L // _NW
_CHUNK = 32
_NCHUNK = _ROWS_PER_W // _CHUNK
_ITERS = _NCHUNK * _B
_VECS_PER_ROW = _H // 16


def _body(x_hbm, pos_hbm, out_hbm, xbuf0, xbuf1, posbuf,
          lsem0, lsem1, ssem0, ssem1):
    wid = lax.axis_index("s") * _NC + lax.axis_index("c")
    row0 = wid * _ROWS_PER_W
    xbufs = (xbuf0, xbuf1)
    lsems = (lsem0, lsem1)
    ssems = (ssem0, ssem1)

    def add_rows(xb):
        def one_row(i, _):
            for u in range(_VECS_PER_ROW):
                sl = pl.ds(u * 16, 16)
                xb[i, sl] = xb[i, sl] + posbuf[i, sl]
            return 0
        lax.fori_loop(0, _CHUNK, one_row, 0)

    def load(k):
        c, b = k // _B, k % _B
        p = k % 2
        cp = pltpu.make_async_copy(
            x_hbm.at[b, pl.ds(row0 + c * _CHUNK, _CHUNK)], xbufs[p], lsems[p])
        cp.start()
        return cp

    def store(k):
        c, b = k // _B, k % _B
        p = k % 2
        cp = pltpu.make_async_copy(
            xbufs[p], out_hbm.at[b, pl.ds(row0 + c * _CHUNK, _CHUNK)], ssems[p])
        cp.start()
        return cp

    loads = {0: load(0)}
    stores = {}
    for k in range(_ITERS):
        p = k % 2
        if k + 1 < _ITERS:
            if k - 1 in stores:
                stores.pop(k - 1).wait()
            loads[k + 1] = load(k + 1)
        if k % _B == 0:
            pltpu.sync_copy(
                pos_hbm.at[pl.ds(row0 + (k // _B) * _CHUNK, _CHUNK)], posbuf)
        loads.pop(k).wait()
        add_rows(xbufs[p])
        stores[k] = store(k)
    stores.pop(_ITERS - 2).wait()
    stores.pop(_ITERS - 1).wait()


def kernel(x, pos_table):
    mesh = plsc.VectorSubcoreMesh(core_axis_name="c", subcore_axis_name="s")
    k = functools.partial(
        pl.kernel,
        mesh=mesh,
        out_type=jax.ShapeDtypeStruct((_B, _L, _H), jnp.float32),
        scratch_types=[
            pltpu.VMEM((_CHUNK, _H), jnp.float32),
            pltpu.VMEM((_CHUNK, _H), jnp.float32),
            pltpu.VMEM((_CHUNK, _H), jnp.float32),
            pltpu.SemaphoreType.DMA,
            pltpu.SemaphoreType.DMA,
            pltpu.SemaphoreType.DMA,
            pltpu.SemaphoreType.DMA,
        ],
    )(_body)
    return k(x, pos_table)


# speedup vs baseline: 3.0648x; 2.4553x over previous
import jax
import jax.numpy as jnp
from jax.experimental import pallas as pl
from jax.experimental.pallas import tpu as pltpu


_BLK_L = 512


def _add_body(x_ref, pos_ref, out_ref):
    out_ref[...] = x_ref[...] + pos_ref[...][None, :, :]


def kernel(x, pos_table):
    B, L, H = x.shape
    blk = _BLK_L
    grid = (L // blk,)
    return pl.pallas_call(
        _add_body,
        grid=grid,
        in_specs=[
            pl.BlockSpec((B, blk, H), lambda i: (0, i, 0)),
            pl.BlockSpec((blk, H), lambda i: (i, 0)),
        ],
        out_specs=pl.BlockSpec((B, blk, H), lambda i: (0, i, 0)),
        out_shape=jax.ShapeDtypeStruct((B, L, H), x.dtype),
        compiler_params=pltpu.CompilerParams(
            dimension_semantics=("parallel",),
        ),
    )(x, pos_table)
